# Initial kernel scaffold; baseline (speedup 1.0000x reference)
import jax
import jax.numpy as jnp
from jax.experimental import pallas as pl


def kernel(edge_index, node_latent, current_velocity, edge_forces, edge_torques, edge_constraints, mass_W1, mass_b1, mass_W2, mass_b2, inertia_W1, inertia_b1, inertia_W2, inertia_b2, ext_W1, ext_b1, ext_W2, ext_b2, vel_W1, vel_b1, vel_W2, vel_b2):
    raise NotImplementedError("write your pallas kernel here")


# speedup vs baseline: 1.3806x; 1.3806x over previous
import functools

import jax
import jax.numpy as jnp
from jax import lax
from jax.experimental import pallas as pl
from jax.experimental.pallas import tpu as pltpu
from jax.experimental.pallas import tpu_sc as plsc

N = 100000
E = 3200000
D = 128

NC = 2
NS = 16
NW = NC * NS
CHUNK = 1024
SUB = 512
NSUB = CHUNK // SUB
NCHUNKS = E // CHUNK
TRIP_LO = NCHUNKS // NW
EXTRA = NCHUNKS - TRIP_LO * NW
ROWS = 6256
ROWS_LAST = N - 15 * ROWS


def _sc_segment_sums(receivers2d, forces, torques, cons, zeros16, pre16):
    mesh = plsc.VectorSubcoreMesh(core_axis_name="c", subcore_axis_name="s",
                                  num_cores=NC, num_subcores=NS)

    @functools.partial(
        pl.kernel,
        out_type=jax.ShapeDtypeStruct((NC, N, 16), jnp.float32),
        mesh=mesh,
        compiler_params=pltpu.CompilerParams(use_tc_tiling_on_sc=False),
        scratch_types=[
            pltpu.VMEM_SHARED((N, 16), jnp.float32),
            pltpu.VMEM((NSUB, SUB), jnp.int32),
            pltpu.VMEM((CHUNK, 16), jnp.float32),
        ],
    )
    def k(recv_hbm, f_hbm, t_hbm, c_hbm, z16_hbm, pre_hbm, out_hbm,
          acc, idx_v, vv):
        c = lax.axis_index("c")
        s = lax.axis_index("s")
        wid = s * NC + c
        r0 = s * ROWS

        @pl.when(s < NS - 1)
        def _():
            pltpu.sync_copy(z16_hbm, acc.at[pl.ds(r0, ROWS)])

        @pl.when(s == NS - 1)
        def _():
            pltpu.sync_copy(z16_hbm.at[pl.ds(0, ROWS_LAST)],
                            acc.at[pl.ds(r0, ROWS_LAST)])

        pltpu.sync_copy(pre_hbm, vv)
        plsc.subcore_barrier()

        trip = jnp.where(wid < EXTRA, TRIP_LO + 1, TRIP_LO)

        def body(kk, carry):
            g = wid + NW * kk
            pltpu.sync_copy(recv_hbm.at[pl.ds(g * NSUB, NSUB)], idx_v)
            pltpu.sync_copy(f_hbm.at[pl.ds(g * CHUNK, CHUNK)],
                            vv.at[:, pl.ds(0, 3)])
            pltpu.sync_copy(t_hbm.at[pl.ds(g * CHUNK, CHUNK)],
                            vv.at[:, pl.ds(3, 3)])
            pltpu.sync_copy(c_hbm.at[pl.ds(g * CHUNK, CHUNK)],
                            vv.at[:, pl.ds(8, 3)])
            for j in range(NSUB):
                pltpu.sync_copy(vv.at[pl.ds(j * SUB, SUB)],
                                acc.at[idx_v.at[j]], add=True)
            return carry

        lax.fori_loop(0, trip, body, 0)
        plsc.subcore_barrier()

        @pl.when(s < NS - 1)
        def _():
            pltpu.sync_copy(acc.at[pl.ds(r0, ROWS)],
                            out_hbm.at[c, pl.ds(r0, ROWS)])

        @pl.when(s == NS - 1)
        def _():
            pltpu.sync_copy(acc.at[pl.ds(r0, ROWS_LAST)],
                            out_hbm.at[c, pl.ds(r0, ROWS_LAST)])

    return k(receivers2d, forces, torques, cons, zeros16, pre16)


BT = 2000


def _tc_body(x_ref, cv_ref, p_ref,
             mw1, mb1, mw2, mb2, iw1, ib1, iw2, ib2,
             ew1, eb1, ew2, eb2, vw1, vb1, vw2, vb2,
             dv_ref, dav_ref, disp_ref):
    x = x_ref[...]

    def mlp(w1, b1, w2, b2):
        h = jnp.maximum(
            jnp.dot(x, w1[...], preferred_element_type=jnp.float32) + b1[...],
            0.0)
        return jnp.dot(h, w2[...], preferred_element_type=jnp.float32) + b2[...]

    inv_mass = mlp(mw1, mb1, mw2, mb2)
    inv_inertia = mlp(iw1, ib1, iw2, ib2)
    dv_ext = mlp(ew1, eb1, ew2, eb2)
    vel_scale = mlp(vw1, vb1, vw2, vb2)

    p = p_ref[0] + p_ref[1]
    net_f = p[:, 0:3]
    net_t = p[:, 3:6]
    cnt = p[:, 6:7]
    sums = p[:, 8:11]

    dv_ref[...] = inv_mass * net_f
    dav_ref[...] = inv_inertia * net_t
    corr = sums / jnp.maximum(cnt, 1.0)
    disp_ref[...] = (cv_ref[...] + dv_ext) * vel_scale + corr


def _tc_decode(node_latent, cur_vel, p16, weights):
    nblk = N // BT
    full = lambda i: (0, 0)
    specs = [
        pl.BlockSpec((BT, D), lambda i: (i, 0)),
        pl.BlockSpec((BT, 3), lambda i: (i, 0)),
        pl.BlockSpec((NC, BT, 16), lambda i: (0, i, 0)),
    ]
    for od in (1, 1, 3, 1):
        specs += [
            pl.BlockSpec((D, D), full),
            pl.BlockSpec((1, D), full),
            pl.BlockSpec((D, od), full),
            pl.BlockSpec((1, od), full),
        ]
    out3 = jax.ShapeDtypeStruct((N, 3), jnp.float32)
    return pl.pallas_call(
        _tc_body,
        grid=(nblk,),
        in_specs=specs,
        out_specs=[pl.BlockSpec((BT, 3), lambda i: (i, 0))] * 3,
        out_shape=[out3, out3, out3],
    )(node_latent, cur_vel, p16, *weights)


def kernel(edge_index, node_latent, current_velocity, edge_forces,
           edge_torques, edge_constraints, mass_W1, mass_b1, mass_W2, mass_b2,
           inertia_W1, inertia_b1, inertia_W2, inertia_b2, ext_W1, ext_b1,
           ext_W2, ext_b2, vel_W1, vel_b1, vel_W2, vel_b2):
    receivers2d = edge_index[1].reshape(E // SUB, SUB)
    zeros16 = jnp.zeros((ROWS, 16), jnp.float32)
    pre16 = jnp.zeros((CHUNK, 16), jnp.float32).at[:, 6].set(1.0)

    p16 = _sc_segment_sums(receivers2d, edge_forces, edge_torques,
                           edge_constraints, zeros16, pre16)

    weights = (
        mass_W1, mass_b1.reshape(1, D), mass_W2, mass_b2.reshape(1, 1),
        inertia_W1, inertia_b1.reshape(1, D), inertia_W2,
        inertia_b2.reshape(1, 1),
        ext_W1, ext_b1.reshape(1, D), ext_W2, ext_b2.reshape(1, 3),
        vel_W1, vel_b1.reshape(1, D), vel_W2, vel_b2.reshape(1, 1),
    )
    return tuple(_tc_decode(node_latent, current_velocity, p16, weights))


# speedup vs baseline: 1.3872x; 1.0048x over previous
import functools

import jax
import jax.numpy as jnp
from jax import lax
from jax.experimental import pallas as pl
from jax.experimental.pallas import tpu as pltpu
from jax.experimental.pallas import tpu_sc as plsc

N = 100000
E = 3200000
D = 128

NC = 2
NS = 16
NW = NC * NS
CHUNK = 512
SUB = 512
NCHUNKS = E // CHUNK
TRIP_LO = NCHUNKS // NW
EXTRA = NCHUNKS - TRIP_LO * NW
PAIRS = (TRIP_LO - 1) // 2
ROWS = 6256
ROWS_LAST = N - 15 * ROWS


def _sc_segment_sums(receivers2d, forces, torques, cons, zeros16, pre16):
    mesh = plsc.VectorSubcoreMesh(core_axis_name="c", subcore_axis_name="s",
                                  num_cores=NC, num_subcores=NS)

    @functools.partial(
        pl.kernel,
        out_type=jax.ShapeDtypeStruct((NC, N, 16), jnp.float32),
        mesh=mesh,
        compiler_params=pltpu.CompilerParams(use_tc_tiling_on_sc=False),
        scratch_types=[
            pltpu.VMEM_SHARED((N, 16), jnp.float32),
            pltpu.VMEM((1, SUB), jnp.int32),
            pltpu.VMEM((1, SUB), jnp.int32),
            pltpu.VMEM((CHUNK, 16), jnp.float32),
            pltpu.VMEM((CHUNK, 16), jnp.float32),
            pltpu.SemaphoreType.DMA,
            pltpu.SemaphoreType.DMA,
        ],
    )
    def k(recv_hbm, f_hbm, t_hbm, c_hbm, z16_hbm, pre_hbm, out_hbm,
          acc, idxA, idxB, vvA, vvB, sem_in, sem_sc):
        c = lax.axis_index("c")
        s = lax.axis_index("s")
        wid = s * NC + c
        r0 = s * ROWS

        @pl.when(s < NS - 1)
        def _():
            pltpu.sync_copy(z16_hbm, acc.at[pl.ds(r0, ROWS)])

        @pl.when(s == NS - 1)
        def _():
            pltpu.sync_copy(z16_hbm.at[pl.ds(0, ROWS_LAST)],
                            acc.at[pl.ds(r0, ROWS_LAST)])

        pltpu.sync_copy(pre_hbm, vvA)
        pltpu.sync_copy(pre_hbm, vvB)
        plsc.subcore_barrier()

        def stage_start(g, idx_r, vv_r):
            pltpu.async_copy(recv_hbm.at[pl.ds(g, 1)], idx_r, sem_in)
            pltpu.async_copy(f_hbm.at[pl.ds(g * CHUNK, CHUNK)],
                             vv_r.at[:, pl.ds(0, 3)], sem_in)
            pltpu.async_copy(t_hbm.at[pl.ds(g * CHUNK, CHUNK)],
                             vv_r.at[:, pl.ds(3, 3)], sem_in)
            pltpu.async_copy(c_hbm.at[pl.ds(g * CHUNK, CHUNK)],
                             vv_r.at[:, pl.ds(8, 3)], sem_in)

        def stage_wait():
            pltpu.make_async_copy(recv_hbm.at[pl.ds(0, 1)], idxA,
                                  sem_in).wait()
            pltpu.make_async_copy(f_hbm.at[pl.ds(0, CHUNK)],
                                  vvA.at[:, pl.ds(0, 3)], sem_in).wait()
            pltpu.make_async_copy(f_hbm.at[pl.ds(0, CHUNK)],
                                  vvA.at[:, pl.ds(3, 3)], sem_in).wait()
            pltpu.make_async_copy(f_hbm.at[pl.ds(0, CHUNK)],
                                  vvA.at[:, pl.ds(8, 3)], sem_in).wait()

        def fire(idx_r, vv_r):
            pltpu.async_copy(vv_r, acc.at[idx_r.at[0]], sem_sc, add=True)

        def drain():
            pltpu.make_async_copy(vvA, acc.at[idxA.at[0]], sem_sc).wait()

        stage_start(wid, idxA, vvA)

        def body(m, carry):
            k1 = 2 * m + 1
            stage_wait()

            @pl.when(m > 0)
            def _():
                drain()

            stage_start(wid + NW * k1, idxB, vvB)
            fire(idxA, vvA)
            stage_wait()
            drain()
            stage_start(wid + NW * (k1 + 1), idxA, vvA)
            fire(idxB, vvB)
            return carry

        lax.fori_loop(0, PAIRS, body, 0)

        stage_wait()
        drain()

        @pl.when(wid < EXTRA)
        def _():
            stage_start(wid + NW * TRIP_LO, idxB, vvB)

        fire(idxA, vvA)

        @pl.when(wid < EXTRA)
        def _():
            stage_wait()
            drain()
            fire(idxB, vvB)
            drain()

        @pl.when(wid >= EXTRA)
        def _():
            drain()

        plsc.subcore_barrier()

        @pl.when(s < NS - 1)
        def _():
            pltpu.sync_copy(acc.at[pl.ds(r0, ROWS)],
                            out_hbm.at[c, pl.ds(r0, ROWS)])

        @pl.when(s == NS - 1)
        def _():
            pltpu.sync_copy(acc.at[pl.ds(r0, ROWS_LAST)],
                            out_hbm.at[c, pl.ds(r0, ROWS_LAST)])

    return k(receivers2d, forces, torques, cons, zeros16, pre16)


BT = 2000


def _tc_body(x_ref, cv_ref, p_ref,
             mw1, mb1, mw2, mb2, iw1, ib1, iw2, ib2,
             ew1, eb1, ew2, eb2, vw1, vb1, vw2, vb2,
             dv_ref, dav_ref, disp_ref):
    x = x_ref[...]

    def mlp(w1, b1, w2, b2):
        h = jnp.maximum(
            jnp.dot(x, w1[...], preferred_element_type=jnp.float32) + b1[...],
            0.0)
        return jnp.dot(h, w2[...], preferred_element_type=jnp.float32) + b2[...]

    inv_mass = mlp(mw1, mb1, mw2, mb2)
    inv_inertia = mlp(iw1, ib1, iw2, ib2)
    dv_ext = mlp(ew1, eb1, ew2, eb2)
    vel_scale = mlp(vw1, vb1, vw2, vb2)

    p = p_ref[0] + p_ref[1]
    net_f = p[:, 0:3]
    net_t = p[:, 3:6]
    cnt = p[:, 6:7]
    sums = p[:, 8:11]

    dv_ref[...] = inv_mass * net_f
    dav_ref[...] = inv_inertia * net_t
    corr = sums / jnp.maximum(cnt, 1.0)
    disp_ref[...] = (cv_ref[...] + dv_ext) * vel_scale + corr


def _tc_decode(node_latent, cur_vel, p16, weights):
    nblk = N // BT
    full = lambda i: (0, 0)
    specs = [
        pl.BlockSpec((BT, D), lambda i: (i, 0)),
        pl.BlockSpec((BT, 3), lambda i: (i, 0)),
        pl.BlockSpec((NC, BT, 16), lambda i: (0, i, 0)),
    ]
    for od in (1, 1, 3, 1):
        specs += [
            pl.BlockSpec((D, D), full),
            pl.BlockSpec((1, D), full),
            pl.BlockSpec((D, od), full),
            pl.BlockSpec((1, od), full),
        ]
    out3 = jax.ShapeDtypeStruct((N, 3), jnp.float32)
    return pl.pallas_call(
        _tc_body,
        grid=(nblk,),
        in_specs=specs,
        out_specs=[pl.BlockSpec((BT, 3), lambda i: (i, 0))] * 3,
        out_shape=[out3, out3, out3],
    )(node_latent, cur_vel, p16, *weights)


def kernel(edge_index, node_latent, current_velocity, edge_forces,
           edge_torques, edge_constraints, mass_W1, mass_b1, mass_W2, mass_b2,
           inertia_W1, inertia_b1, inertia_W2, inertia_b2, ext_W1, ext_b1,
           ext_W2, ext_b2, vel_W1, vel_b1, vel_W2, vel_b2):
    receivers2d = edge_index[1].reshape(E // SUB, SUB)
    zeros16 = jnp.zeros((ROWS, 16), jnp.float32)
    pre16 = jnp.zeros((CHUNK, 16), jnp.float32).at[:, 6].set(1.0)

    p16 = _sc_segment_sums(receivers2d, edge_forces, edge_torques,
                           edge_constraints, zeros16, pre16)

    weights = (
        mass_W1, mass_b1.reshape(1, D), mass_W2, mass_b2.reshape(1, 1),
        inertia_W1, inertia_b1.reshape(1, D), inertia_W2,
        inertia_b2.reshape(1, 1),
        ext_W1, ext_b1.reshape(1, D), ext_W2, ext_b2.reshape(1, 3),
        vel_W1, vel_b1.reshape(1, D), vel_W2, vel_b2.reshape(1, 1),
    )
    return tuple(_tc_decode(node_latent, current_velocity, p16, weights))


# speedup vs baseline: 1.5483x; 1.1161x over previous
import functools

import jax
import jax.numpy as jnp
from jax import lax
from jax.experimental import pallas as pl
from jax.experimental.pallas import tpu as pltpu
from jax.experimental.pallas import tpu_sc as plsc

N = 100000
E = 3200000
D = 128

NC = 2
NS = 16
NW = NC * NS
CHUNK = 512
SUB = 512
NCHUNKS = E // CHUNK
TRIP_LO = NCHUNKS // NW
EXTRA = NCHUNKS - TRIP_LO * NW
PAIRS = (TRIP_LO - 1) // 2
ROWS = 6256
ROWS_LAST = N - 15 * ROWS


def _sc_segment_sums(receivers2d, cols9, zeros16, pre16):
    mesh = plsc.VectorSubcoreMesh(core_axis_name="c", subcore_axis_name="s",
                                  num_cores=NC, num_subcores=NS)

    @functools.partial(
        pl.kernel,
        out_type=jax.ShapeDtypeStruct((NC, N, 16), jnp.float32),
        mesh=mesh,
        compiler_params=pltpu.CompilerParams(use_tc_tiling_on_sc=False),
        scratch_types=[
            pltpu.VMEM_SHARED((N, 16), jnp.float32),
            pltpu.VMEM((1, SUB), jnp.int32),
            pltpu.VMEM((1, SUB), jnp.int32),
            pltpu.VMEM((CHUNK, 16), jnp.float32),
            pltpu.VMEM((CHUNK, 16), jnp.float32),
            pltpu.SemaphoreType.DMA,
            pltpu.SemaphoreType.DMA,
        ],
    )
    def k(recv_hbm, c0, c1, c2, c3, c4, c5, c6, c7, c8, z16_hbm, pre_hbm,
          out_hbm, acc, idxA, idxB, vvA, vvB, sem_in, sem_sc):
        cols = (c0, c1, c2, c3, c4, c5, c6, c7, c8)
        c = lax.axis_index("c")
        s = lax.axis_index("s")
        wid = s * NC + c
        r0 = s * ROWS

        @pl.when(s < NS - 1)
        def _():
            pltpu.sync_copy(z16_hbm, acc.at[pl.ds(r0, ROWS)])

        @pl.when(s == NS - 1)
        def _():
            pltpu.sync_copy(z16_hbm.at[pl.ds(0, ROWS_LAST)],
                            acc.at[pl.ds(r0, ROWS_LAST)])

        pltpu.sync_copy(pre_hbm, vvA)
        pltpu.sync_copy(pre_hbm, vvB)
        plsc.subcore_barrier()

        DSTS = (0, 1, 2, 3, 4, 5, 8, 9, 10)

        def stage_start(g, idx_r, vv_r):
            pltpu.async_copy(recv_hbm.at[pl.ds(g, 1)], idx_r, sem_in)
            for col, dst in zip(cols, DSTS):
                pltpu.async_copy(col.at[pl.ds(g * CHUNK, CHUNK)],
                                 vv_r.at[:, pl.ds(dst, 1)], sem_in)

        def stage_wait():
            pltpu.make_async_copy(recv_hbm.at[pl.ds(0, 1)], idxA,
                                  sem_in).wait()
            for col, dst in zip(cols, DSTS):
                pltpu.make_async_copy(col.at[pl.ds(0, CHUNK)],
                                      vvA.at[:, pl.ds(dst, 1)], sem_in).wait()

        def fire(idx_r, vv_r):
            pltpu.async_copy(vv_r, acc.at[idx_r.at[0]], sem_sc, add=True)

        def drain():
            pltpu.make_async_copy(vvA, acc.at[idxA.at[0]], sem_sc).wait()

        stage_start(wid, idxA, vvA)

        def body(m, carry):
            k1 = 2 * m + 1
            stage_wait()

            @pl.when(m > 0)
            def _():
                drain()

            stage_start(wid + NW * k1, idxB, vvB)
            fire(idxA, vvA)
            stage_wait()
            drain()
            stage_start(wid + NW * (k1 + 1), idxA, vvA)
            fire(idxB, vvB)
            return carry

        lax.fori_loop(0, PAIRS, body, 0)

        stage_wait()
        drain()

        @pl.when(wid < EXTRA)
        def _():
            stage_start(wid + NW * TRIP_LO, idxB, vvB)

        fire(idxA, vvA)

        @pl.when(wid < EXTRA)
        def _():
            stage_wait()
            drain()
            fire(idxB, vvB)
            drain()

        @pl.when(wid >= EXTRA)
        def _():
            drain()

        plsc.subcore_barrier()

        @pl.when(s < NS - 1)
        def _():
            pltpu.sync_copy(acc.at[pl.ds(r0, ROWS)],
                            out_hbm.at[c, pl.ds(r0, ROWS)])

        @pl.when(s == NS - 1)
        def _():
            pltpu.sync_copy(acc.at[pl.ds(r0, ROWS_LAST)],
                            out_hbm.at[c, pl.ds(r0, ROWS_LAST)])

    return k(receivers2d, *cols9, zeros16, pre16)


BT = 2000


def _tc_body(x_ref, cv_ref, p_ref,
             mw1, mb1, mw2, mb2, iw1, ib1, iw2, ib2,
             ew1, eb1, ew2, eb2, vw1, vb1, vw2, vb2,
             dv_ref, dav_ref, disp_ref):
    x = x_ref[...]

    def mlp(w1, b1, w2, b2):
        h = jnp.maximum(
            jnp.dot(x, w1[...], preferred_element_type=jnp.float32) + b1[...],
            0.0)
        return jnp.dot(h, w2[...], preferred_element_type=jnp.float32) + b2[...]

    inv_mass = mlp(mw1, mb1, mw2, mb2)
    inv_inertia = mlp(iw1, ib1, iw2, ib2)
    dv_ext = mlp(ew1, eb1, ew2, eb2)
    vel_scale = mlp(vw1, vb1, vw2, vb2)

    p = p_ref[0] + p_ref[1]
    net_f = p[:, 0:3]
    net_t = p[:, 3:6]
    cnt = p[:, 6:7]
    sums = p[:, 8:11]

    dv_ref[...] = inv_mass * net_f
    dav_ref[...] = inv_inertia * net_t
    corr = sums / jnp.maximum(cnt, 1.0)
    disp_ref[...] = (cv_ref[...] + dv_ext) * vel_scale + corr


def _tc_decode(node_latent, cur_vel, p16, weights):
    nblk = N // BT
    full = lambda i: (0, 0)
    specs = [
        pl.BlockSpec((BT, D), lambda i: (i, 0)),
        pl.BlockSpec((BT, 3), lambda i: (i, 0)),
        pl.BlockSpec((NC, BT, 16), lambda i: (0, i, 0)),
    ]
    for od in (1, 1, 3, 1):
        specs += [
            pl.BlockSpec((D, D), full),
            pl.BlockSpec((1, D), full),
            pl.BlockSpec((D, od), full),
            pl.BlockSpec((1, od), full),
        ]
    out3 = jax.ShapeDtypeStruct((N, 3), jnp.float32)
    return pl.pallas_call(
        _tc_body,
        grid=(nblk,),
        in_specs=specs,
        out_specs=[pl.BlockSpec((BT, 3), lambda i: (i, 0))] * 3,
        out_shape=[out3, out3, out3],
    )(node_latent, cur_vel, p16, *weights)


def kernel(edge_index, node_latent, current_velocity, edge_forces,
           edge_torques, edge_constraints, mass_W1, mass_b1, mass_W2, mass_b2,
           inertia_W1, inertia_b1, inertia_W2, inertia_b2, ext_W1, ext_b1,
           ext_W2, ext_b2, vel_W1, vel_b1, vel_W2, vel_b2):
    receivers2d = edge_index[1].reshape(E // SUB, SUB)
    zeros16 = jnp.zeros((ROWS, 16), jnp.float32)
    pre16 = jnp.zeros((CHUNK, 16), jnp.float32).at[:, 6].set(1.0)
    cols9 = tuple(a[:, i:i + 1] for a in (edge_forces, edge_torques,
                                          edge_constraints) for i in range(3))

    p16 = _sc_segment_sums(receivers2d, cols9, zeros16, pre16)

    weights = (
        mass_W1, mass_b1.reshape(1, D), mass_W2, mass_b2.reshape(1, 1),
        inertia_W1, inertia_b1.reshape(1, D), inertia_W2,
        inertia_b2.reshape(1, 1),
        ext_W1, ext_b1.reshape(1, D), ext_W2, ext_b2.reshape(1, 3),
        vel_W1, vel_b1.reshape(1, D), vel_W2, vel_b2.reshape(1, 1),
    )
    return tuple(_tc_decode(node_latent, current_velocity, p16, weights))
